# Initial kernel scaffold; baseline (speedup 1.0000x reference)
import jax
import jax.numpy as jnp
from jax.experimental import pallas as pl


def kernel(lidar_mu, lidar_log_sigma, r1_mu, r1_log_sigma, r2_mu, r2_log_sigma, x_t, x_tp1, pose_by_frame, frame_id_t, batch_id_t, sensor_id_t, batch_id_tp1, sensor_id_tp1):
    raise NotImplementedError("write your pallas kernel here")


# speedup vs baseline: 3.2666x; 3.2666x over previous
import functools

import jax
import jax.numpy as jnp
from jax.experimental import pallas as pl

_WINDOW = 4
_IDX_VR = 6
_ASSOC_TOPK = 5
_ASSOC_TAU = 0.5
_GATE_LIDAR = 0.15
_GATE_RADAR = 0.3
_RADAR_LOSS_WEIGHT = 5.0
_REG_LAMBDA = 1e-3


def _se2_apply(p, pts):
    x, y, th = p[0], p[1], p[2]
    c, s = jnp.cos(th), jnp.sin(th)
    R = jnp.stack([jnp.stack([c, -s]), jnp.stack([s, c])])
    return pts @ R.T + jnp.stack([x, y])


def _se2_inv(p):
    x, y, th = p[0], p[1], p[2]
    c, s = jnp.cos(th), jnp.sin(th)
    return jnp.stack([-(c * x + s * y), -(-s * x + c * y), -th])


def _loss_block_kernel(nb, lmu_ref, lls_ref, rxy_ref, r1m_ref, r1s_ref,
                       r2m_ref, r2s_ref, fid_ref, bid_ref, sid_ref,
                       gx_refs_ref, gy_refs_ref, kvr_ref, kbid_ref,
                       ksid_ref, out_ref):
    f32 = jnp.float32
    nk = kvr_ref.shape[1]
    tau_sq = max(_ASSOC_TAU ** 2, 1e-6)
    t_idx = _WINDOW - 1

    lmu = lmu_ref[...]
    lls = lls_ref[...]
    rxy = rxy_ref[...]
    fid = fid_ref[...]
    bid = bid_ref[...]
    sid = sid_ref[...]
    kvr = kvr_ref[...]
    kbid = kbid_ref[...]
    ksid = ksid_ref[...]
    gxs = [gx_refs_ref[b:b + 1, :] for b in range(nb)]
    gys = [gy_refs_ref[b:b + 1, :] for b in range(nb)]

    def r8(x):
        p = x * 65537.0
        return p - (p - x)

    b0 = bid == 0
    b2s = [gxs[b] * gxs[b] + gys[b] * gys[b] for b in range(nb)]
    gxh = [r8(g) for g in gxs]
    gyh = [r8(g) for g in gys]
    if nb == 2:
        gx_sel = jnp.where(b0, gxs[0], gxs[1])
        gy_sel = jnp.where(b0, gys[0], gys[1])
        gxh_sel = jnp.where(b0, gxh[0], gxh[1])
        gyh_sel = jnp.where(b0, gyh[0], gyh[1])
        b2_sel = jnp.where(b0, b2s[0], b2s[1])
    else:
        gx_sel = jnp.broadcast_to(gxs[0], (lmu.shape[0], nk))
        gy_sel = jnp.broadcast_to(gys[0], (lmu.shape[0], nk))
        gxh_sel = r8(gx_sel)
        gyh_sel = r8(gy_sel)
        b2_sel = jnp.broadcast_to(b2s[0], (lmu.shape[0], nk))

    is_lidar = sid == 0
    qx = jnp.where(is_lidar, lmu[:, 0:1], rxy[:, 0:1])
    qy = jnp.where(is_lidar, lmu[:, 1:2], rxy[:, 1:2])

    a2 = qx * qx + qy * qy
    ab = r8(qx) * gxh_sel + r8(qy) * gyh_sel
    sq = jnp.maximum(a2 + b2_sel - 2.0 * ab, 0.0)
    d = jnp.sqrt(sq)

    col_ok = (kbid == bid) & (ksid == sid)
    inf = jnp.float32(jnp.inf)
    running = jnp.where(col_ok, d, inf)

    iota = jax.lax.broadcasted_iota(jnp.int32, (1, nk), 1)
    ms = []
    gxk = []
    gyk = []
    vr0 = None
    for k in range(_ASSOC_TOPK):
        m = jnp.min(running, axis=1, keepdims=True)
        hit = running == m
        idx = jnp.min(jnp.where(hit, iota, nk), axis=1, keepdims=True)
        one = iota == idx
        gxk.append(jnp.sum(jnp.where(one, gx_sel, 0.0), axis=1, keepdims=True))
        gyk.append(jnp.sum(jnp.where(one, gy_sel, 0.0), axis=1, keepdims=True))
        if k == 0:
            vr0 = jnp.sum(jnp.where(one, kvr, 0.0), axis=1, keepdims=True)
        ms.append(m)
        if k + 1 < _ASSOC_TOPK:
            running = jnp.where(one, inf, running)

    z = [-(m * m) / tau_sq for m in ms]
    z0 = z[0]
    es = [jnp.exp(zk - z0) for zk in z]
    denom = ((es[0] + es[1]) + es[2]) + (es[3] + es[4])
    ex = jnp.zeros_like(z0)
    ey = jnp.zeros_like(z0)
    for k in range(_ASSOC_TOPK):
        w = es[k] / denom
        ex = ex + w * gxk[k]
        ey = ey + w * gyk[k]

    var0 = jnp.exp(2.0 * lls[:, 0:1])
    var1 = jnp.exp(2.0 * lls[:, 1:2])
    nll_l = 0.5 * ((2.0 * lls[:, 0:1] + (ex - lmu[:, 0:1]) ** 2 / (var0 + 1e-12))
                   + (2.0 * lls[:, 1:2] + (ey - lmu[:, 1:2]) ** 2 / (var1 + 1e-12)))

    s1 = sid == 1
    rmu = jnp.where(s1, r1m_ref[...], r2m_ref[...])
    rls = jnp.where(s1, r1s_ref[...], r2s_ref[...])
    rvar = jnp.exp(2.0 * rls)
    nll_r = 0.5 * (2.0 * rls + (vr0 - rmu) ** 2 / (rvar + 1e-12))

    is_t = fid == t_idx
    m0 = ms[0]
    v_l = (m0 <= _GATE_LIDAR) & is_lidar & is_t
    v_r = (m0 <= _GATE_RADAR) & is_t

    parts = []
    for b in range(nb):
        bm = bid == b
        vlb = v_l & bm
        parts.append(jnp.sum(jnp.where(vlb, nll_l, 0.0)))
        parts.append(jnp.sum(vlb.astype(f32)))
        for s in (1, 2):
            vrb = v_r & bm & (sid == s)
            parts.append(jnp.sum(jnp.where(vrb, nll_r, 0.0)))
            parts.append(jnp.sum(vrb.astype(f32)))
    parts.append(jnp.sum(lls * lls))
    parts.append(jnp.sum(r1s_ref[...] ** 2))
    parts.append(jnp.sum(r2s_ref[...] ** 2))
    out_ref[...] = jnp.stack(parts).reshape(1, 1, len(parts))


def kernel(lidar_mu, lidar_log_sigma, r1_mu, r1_log_sigma, r2_mu,
           r2_log_sigma, x_t, x_tp1, pose_by_frame, frame_id_t, batch_id_t,
           sensor_id_t, batch_id_tp1, sensor_id_tp1):
    n_t = lidar_mu.shape[0]
    nk = x_tp1.shape[0]
    nb = pose_by_frame.shape[0]
    t_idx = _WINDOW - 1

    blk = 64
    while n_t % blk:
        blk //= 2
    grid = n_t // blk
    nout = 6 * nb + 3

    f32 = jnp.float32
    i32 = jnp.int32
    rxy = x_t[:, :2].astype(f32)
    kvr = x_tp1[:, _IDX_VR].reshape(1, nk).astype(f32)
    fid = frame_id_t.reshape(n_t, 1).astype(i32)
    bidt = batch_id_t.reshape(n_t, 1).astype(i32)
    sidt = sensor_id_t.reshape(n_t, 1).astype(i32)
    kbid = batch_id_tp1.reshape(1, nk).astype(i32)
    ksid = sensor_id_tp1.reshape(1, nk).astype(i32)

    gt_rows = []
    for b in range(nb):
        pose_t = pose_by_frame[b, t_idx]
        pose_tp1 = pose_by_frame[b, t_idx + 1]
        g = _se2_apply(_se2_inv(pose_t), _se2_apply(pose_tp1, x_tp1[:, :2]))
        gt_rows.append(g)
    gxs = jnp.stack([g[:, 0] for g in gt_rows]).astype(f32)
    gys = jnp.stack([g[:, 1] for g in gt_rows]).astype(f32)

    row = lambda w: pl.BlockSpec((blk, w), lambda i: (i, 0))
    key = pl.BlockSpec((1, nk), lambda i: (0, 0))
    keyb = pl.BlockSpec((nb, nk), lambda i: (0, 0))

    partials = pl.pallas_call(
        functools.partial(_loss_block_kernel, nb),
        grid=(grid,),
        in_specs=[
            row(2), row(2), row(2), row(1), row(1), row(1), row(1),
            row(1), row(1), row(1),
            keyb, keyb, key, key, key,
        ],
        out_specs=pl.BlockSpec((1, 1, nout), lambda i: (i, 0, 0)),
        out_shape=jax.ShapeDtypeStruct((grid, 1, nout), f32),
    )(lidar_mu.astype(f32), lidar_log_sigma.astype(f32), rxy,
      r1_mu.astype(f32), r1_log_sigma.astype(f32), r2_mu.astype(f32),
      r2_log_sigma.astype(f32), fid, bidt, sidt, gxs, gys, kvr, kbid, ksid)

    p = partials.reshape(grid, nout).sum(axis=0)

    zero = jnp.asarray(0.0, f32)

    def seg(s, c):
        return jnp.where(c > 0, s / jnp.maximum(c, 1.0), 0.0)

    lidar_sum = zero
    lidar_cnt = zero
    r1_sum = zero
    r1_cnt = zero
    r2_sum = zero
    r2_cnt = zero
    for b in range(nb):
        o = 6 * b
        lidar_sum = lidar_sum + seg(p[o + 0], p[o + 1])
        lidar_cnt = lidar_cnt + (p[o + 1] > 0).astype(f32)
        r1_sum = r1_sum + seg(p[o + 2], p[o + 3])
        r1_cnt = r1_cnt + (p[o + 3] > 0).astype(f32)
        r2_sum = r2_sum + seg(p[o + 4], p[o + 5])
        r2_cnt = r2_cnt + (p[o + 5] > 0).astype(f32)
    loss_l = jnp.where(lidar_cnt > 0, lidar_sum / jnp.maximum(lidar_cnt, 1.0), zero)
    loss_r1 = jnp.where(r1_cnt > 0, r1_sum / jnp.maximum(r1_cnt, 1.0), zero)
    loss_r2 = jnp.where(r2_cnt > 0, r2_sum / jnp.maximum(r2_cnt, 1.0), zero)
    o = 6 * nb
    reg = (p[o] / (lidar_log_sigma.size) + p[o + 1] / r1_log_sigma.size
           + p[o + 2] / r2_log_sigma.size)
    total = loss_l + _RADAR_LOSS_WEIGHT * (loss_r1 + loss_r2) + _REG_LAMBDA * reg
    return (total, loss_l, loss_r1, loss_r2, reg)


# speedup vs baseline: 3.3060x; 1.0121x over previous
import functools

import jax
import jax.numpy as jnp
from jax.experimental import pallas as pl

_WINDOW = 4
_IDX_VR = 6
_ASSOC_TOPK = 5
_ASSOC_TAU = 0.5
_GATE_LIDAR = 0.15
_GATE_RADAR = 0.3
_RADAR_LOSS_WEIGHT = 5.0
_REG_LAMBDA = 1e-3


def _se2_apply(p, pts):
    x, y, th = p[0], p[1], p[2]
    c, s = jnp.cos(th), jnp.sin(th)
    R = jnp.stack([jnp.stack([c, -s]), jnp.stack([s, c])])
    return pts @ R.T + jnp.stack([x, y])


def _se2_inv(p):
    x, y, th = p[0], p[1], p[2]
    c, s = jnp.cos(th), jnp.sin(th)
    return jnp.stack([-(c * x + s * y), -(-s * x + c * y), -th])


def _loss_block_kernel(nb, lmu_ref, lls_ref, rxy_ref, r1m_ref, r1s_ref,
                       r2m_ref, r2s_ref, fid_ref, bid_ref, sid_ref,
                       gx_refs_ref, gy_refs_ref, kvr_ref, kbid_ref,
                       ksid_ref, out_ref):
    f32 = jnp.float32
    nk = kvr_ref.shape[1]
    tau_sq = max(_ASSOC_TAU ** 2, 1e-6)
    t_idx = _WINDOW - 1

    lmu = lmu_ref[...]
    lls = lls_ref[...]
    rxy = rxy_ref[...]
    fid = fid_ref[...]
    bid = bid_ref[...]
    sid = sid_ref[...]
    kvr = kvr_ref[...]
    kbid = kbid_ref[...]
    ksid = ksid_ref[...]
    gxs = [gx_refs_ref[b:b + 1, :] for b in range(nb)]
    gys = [gy_refs_ref[b:b + 1, :] for b in range(nb)]

    def r8(x):
        p = x * 65537.0
        return p - (p - x)

    b0 = bid == 0
    b2s = [gxs[b] * gxs[b] + gys[b] * gys[b] for b in range(nb)]
    gxh = [r8(g) for g in gxs]
    gyh = [r8(g) for g in gys]
    if nb == 2:
        gx_sel = jnp.where(b0, gxs[0], gxs[1])
        gy_sel = jnp.where(b0, gys[0], gys[1])
        gxh_sel = jnp.where(b0, gxh[0], gxh[1])
        gyh_sel = jnp.where(b0, gyh[0], gyh[1])
        b2_sel = jnp.where(b0, b2s[0], b2s[1])
    else:
        gx_sel = jnp.broadcast_to(gxs[0], (lmu.shape[0], nk))
        gy_sel = jnp.broadcast_to(gys[0], (lmu.shape[0], nk))
        gxh_sel = r8(gx_sel)
        gyh_sel = r8(gy_sel)
        b2_sel = jnp.broadcast_to(b2s[0], (lmu.shape[0], nk))

    is_lidar = sid == 0
    qx = jnp.where(is_lidar, lmu[:, 0:1], rxy[:, 0:1])
    qy = jnp.where(is_lidar, lmu[:, 1:2], rxy[:, 1:2])

    a2 = qx * qx + qy * qy
    ab = r8(qx) * gxh_sel + r8(qy) * gyh_sel
    sq = jnp.maximum(a2 + b2_sel - 2.0 * ab, 0.0)
    d = jnp.sqrt(sq)

    col_ok = (kbid == bid) & (ksid == sid)
    inf = jnp.float32(jnp.inf)
    running = jnp.where(col_ok, d, inf)

    iota = jax.lax.broadcasted_iota(jnp.int32, (1, nk), 1)
    ms = []
    gxk = []
    gyk = []
    vr0 = None
    for k in range(_ASSOC_TOPK):
        m = jnp.min(running, axis=1, keepdims=True)
        hit = running == m
        idx = jnp.min(jnp.where(hit, iota, nk), axis=1, keepdims=True)
        one = iota == idx
        gxk.append(jnp.sum(jnp.where(one, gx_sel, 0.0), axis=1, keepdims=True))
        gyk.append(jnp.sum(jnp.where(one, gy_sel, 0.0), axis=1, keepdims=True))
        if k == 0:
            vr0 = jnp.sum(jnp.where(one, kvr, 0.0), axis=1, keepdims=True)
        ms.append(m)
        if k + 1 < _ASSOC_TOPK:
            running = jnp.where(one, inf, running)

    z = [-(m * m) / tau_sq for m in ms]
    z0 = z[0]
    es = [jnp.exp(zk - z0) for zk in z]
    denom = ((es[0] + es[1]) + es[2]) + (es[3] + es[4])
    ex = jnp.zeros_like(z0)
    ey = jnp.zeros_like(z0)
    for k in range(_ASSOC_TOPK):
        w = es[k] / denom
        ex = ex + w * gxk[k]
        ey = ey + w * gyk[k]

    var0 = jnp.exp(2.0 * lls[:, 0:1])
    var1 = jnp.exp(2.0 * lls[:, 1:2])
    nll_l = 0.5 * ((2.0 * lls[:, 0:1] + (ex - lmu[:, 0:1]) ** 2 / (var0 + 1e-12))
                   + (2.0 * lls[:, 1:2] + (ey - lmu[:, 1:2]) ** 2 / (var1 + 1e-12)))

    s1 = sid == 1
    rmu = jnp.where(s1, r1m_ref[...], r2m_ref[...])
    rls = jnp.where(s1, r1s_ref[...], r2s_ref[...])
    rvar = jnp.exp(2.0 * rls)
    nll_r = 0.5 * (2.0 * rls + (vr0 - rmu) ** 2 / (rvar + 1e-12))

    is_t = fid == t_idx
    m0 = ms[0]
    v_l = (m0 <= _GATE_LIDAR) & is_lidar & is_t
    v_r = (m0 <= _GATE_RADAR) & is_t

    parts = []
    for b in range(nb):
        bm = bid == b
        vlb = v_l & bm
        parts.append(jnp.sum(jnp.where(vlb, nll_l, 0.0)))
        parts.append(jnp.sum(vlb.astype(f32)))
        for s in (1, 2):
            vrb = v_r & bm & (sid == s)
            parts.append(jnp.sum(jnp.where(vrb, nll_r, 0.0)))
            parts.append(jnp.sum(vrb.astype(f32)))
    parts.append(jnp.sum(lls * lls))
    parts.append(jnp.sum(r1s_ref[...] ** 2))
    parts.append(jnp.sum(r2s_ref[...] ** 2))
    out_ref[...] = jnp.stack(parts).reshape(1, 1, len(parts))


def kernel(lidar_mu, lidar_log_sigma, r1_mu, r1_log_sigma, r2_mu,
           r2_log_sigma, x_t, x_tp1, pose_by_frame, frame_id_t, batch_id_t,
           sensor_id_t, batch_id_tp1, sensor_id_tp1):
    n_t = lidar_mu.shape[0]
    nk = x_tp1.shape[0]
    nb = pose_by_frame.shape[0]
    t_idx = _WINDOW - 1

    blk = 128
    while n_t % blk:
        blk //= 2
    grid = n_t // blk
    nout = 6 * nb + 3

    f32 = jnp.float32
    i32 = jnp.int32
    rxy = x_t[:, :2].astype(f32)
    kvr = x_tp1[:, _IDX_VR].reshape(1, nk).astype(f32)
    fid = frame_id_t.reshape(n_t, 1).astype(i32)
    bidt = batch_id_t.reshape(n_t, 1).astype(i32)
    sidt = sensor_id_t.reshape(n_t, 1).astype(i32)
    kbid = batch_id_tp1.reshape(1, nk).astype(i32)
    ksid = sensor_id_tp1.reshape(1, nk).astype(i32)

    gt_rows = []
    for b in range(nb):
        pose_t = pose_by_frame[b, t_idx]
        pose_tp1 = pose_by_frame[b, t_idx + 1]
        g = _se2_apply(_se2_inv(pose_t), _se2_apply(pose_tp1, x_tp1[:, :2]))
        gt_rows.append(g)
    gxs = jnp.stack([g[:, 0] for g in gt_rows]).astype(f32)
    gys = jnp.stack([g[:, 1] for g in gt_rows]).astype(f32)

    row = lambda w: pl.BlockSpec((blk, w), lambda i: (i, 0))
    key = pl.BlockSpec((1, nk), lambda i: (0, 0))
    keyb = pl.BlockSpec((nb, nk), lambda i: (0, 0))

    partials = pl.pallas_call(
        functools.partial(_loss_block_kernel, nb),
        grid=(grid,),
        in_specs=[
            row(2), row(2), row(2), row(1), row(1), row(1), row(1),
            row(1), row(1), row(1),
            keyb, keyb, key, key, key,
        ],
        out_specs=pl.BlockSpec((1, 1, nout), lambda i: (i, 0, 0)),
        out_shape=jax.ShapeDtypeStruct((grid, 1, nout), f32),
    )(lidar_mu.astype(f32), lidar_log_sigma.astype(f32), rxy,
      r1_mu.astype(f32), r1_log_sigma.astype(f32), r2_mu.astype(f32),
      r2_log_sigma.astype(f32), fid, bidt, sidt, gxs, gys, kvr, kbid, ksid)

    p = partials.reshape(grid, nout).sum(axis=0)

    zero = jnp.asarray(0.0, f32)

    def seg(s, c):
        return jnp.where(c > 0, s / jnp.maximum(c, 1.0), 0.0)

    lidar_sum = zero
    lidar_cnt = zero
    r1_sum = zero
    r1_cnt = zero
    r2_sum = zero
    r2_cnt = zero
    for b in range(nb):
        o = 6 * b
        lidar_sum = lidar_sum + seg(p[o + 0], p[o + 1])
        lidar_cnt = lidar_cnt + (p[o + 1] > 0).astype(f32)
        r1_sum = r1_sum + seg(p[o + 2], p[o + 3])
        r1_cnt = r1_cnt + (p[o + 3] > 0).astype(f32)
        r2_sum = r2_sum + seg(p[o + 4], p[o + 5])
        r2_cnt = r2_cnt + (p[o + 5] > 0).astype(f32)
    loss_l = jnp.where(lidar_cnt > 0, lidar_sum / jnp.maximum(lidar_cnt, 1.0), zero)
    loss_r1 = jnp.where(r1_cnt > 0, r1_sum / jnp.maximum(r1_cnt, 1.0), zero)
    loss_r2 = jnp.where(r2_cnt > 0, r2_sum / jnp.maximum(r2_cnt, 1.0), zero)
    o = 6 * nb
    reg = (p[o] / (lidar_log_sigma.size) + p[o + 1] / r1_log_sigma.size
           + p[o + 2] / r2_log_sigma.size)
    total = loss_l + _RADAR_LOSS_WEIGHT * (loss_r1 + loss_r2) + _REG_LAMBDA * reg
    return (total, loss_l, loss_r1, loss_r2, reg)


# speedup vs baseline: 10.1365x; 3.0661x over previous
import functools

import jax
import jax.numpy as jnp
from jax.experimental import pallas as pl

_WINDOW = 4
_IDX_VR = 6
_ASSOC_TOPK = 5
_ASSOC_TAU = 0.5
_GATE_LIDAR = 0.15
_GATE_RADAR = 0.3
_RADAR_LOSS_WEIGHT = 5.0
_REG_LAMBDA = 1e-3


def _se2_apply(p, pts):
    x, y, th = p[0], p[1], p[2]
    c, s = jnp.cos(th), jnp.sin(th)
    R = jnp.stack([jnp.stack([c, -s]), jnp.stack([s, c])])
    return pts @ R.T + jnp.stack([x, y])


def _se2_inv(p):
    x, y, th = p[0], p[1], p[2]
    c, s = jnp.cos(th), jnp.sin(th)
    return jnp.stack([-(c * x + s * y), -(-s * x + c * y), -th])


def _loss_block_kernel(nb, lmu_ref, lls_ref, rxy_ref, r1m_ref, r1s_ref,
                       r2m_ref, r2s_ref, fid_ref, bid_ref, sid_ref,
                       gx_refs_ref, gy_refs_ref, kvr_ref, kbid_ref,
                       ksid_ref, out_ref):
    f32 = jnp.float32
    nk = kvr_ref.shape[1]
    tau_sq = max(_ASSOC_TAU ** 2, 1e-6)
    t_idx = _WINDOW - 1

    lmu = lmu_ref[...]
    lls = lls_ref[...]
    rxy = rxy_ref[...]
    fid = fid_ref[...]
    bid = bid_ref[...]
    sid = sid_ref[...]
    kvr = kvr_ref[...]
    kbid = kbid_ref[...]
    ksid = ksid_ref[...]
    gxs = [gx_refs_ref[b:b + 1, :] for b in range(nb)]
    gys = [gy_refs_ref[b:b + 1, :] for b in range(nb)]

    def r8(x):
        p = x * 65537.0
        return p - (p - x)

    b0 = bid == 0
    b2s = [gxs[b] * gxs[b] + gys[b] * gys[b] for b in range(nb)]
    gxh = [r8(g) for g in gxs]
    gyh = [r8(g) for g in gys]
    if nb == 2:
        gx_sel = jnp.where(b0, gxs[0], gxs[1])
        gy_sel = jnp.where(b0, gys[0], gys[1])
        gxh_sel = jnp.where(b0, gxh[0], gxh[1])
        gyh_sel = jnp.where(b0, gyh[0], gyh[1])
        b2_sel = jnp.where(b0, b2s[0], b2s[1])
    else:
        gx_sel = jnp.broadcast_to(gxs[0], (lmu.shape[0], nk))
        gy_sel = jnp.broadcast_to(gys[0], (lmu.shape[0], nk))
        gxh_sel = r8(gx_sel)
        gyh_sel = r8(gy_sel)
        b2_sel = jnp.broadcast_to(b2s[0], (lmu.shape[0], nk))

    is_lidar = sid == 0
    is_t = fid == t_idx
    nout = 6 * nb + 3

    reg_parts = [jnp.sum(lls * lls), jnp.sum(r1s_ref[...] ** 2),
                 jnp.sum(r2s_ref[...] ** 2)]
    out_ref[...] = jnp.concatenate(
        [jnp.zeros((6 * nb,), f32), jnp.stack(reg_parts)]).reshape(1, 1, nout)

    @pl.when(jnp.any(is_t))
    def _live():
        qx = jnp.where(is_lidar, lmu[:, 0:1], rxy[:, 0:1])
        qy = jnp.where(is_lidar, lmu[:, 1:2], rxy[:, 1:2])

        a2 = qx * qx + qy * qy
        ab = r8(qx) * gxh_sel + r8(qy) * gyh_sel
        sq = jnp.maximum(a2 + b2_sel - 2.0 * ab, 0.0)
        d = jnp.sqrt(sq)

        col_ok = (kbid == bid) & (ksid == sid)
        inf = jnp.float32(jnp.inf)
        running = jnp.where(col_ok, d, inf)

        iota = jax.lax.broadcasted_iota(jnp.int32, (1, nk), 1)
        m0 = jnp.min(running, axis=1, keepdims=True)
        hit0 = running == m0
        idx0 = jnp.min(jnp.where(hit0, iota, nk), axis=1, keepdims=True)
        one0 = iota == idx0
        vr0 = jnp.sum(jnp.where(one0, kvr, 0.0), axis=1, keepdims=True)

        s1 = sid == 1
        rmu = jnp.where(s1, r1m_ref[...], r2m_ref[...])
        rls = jnp.where(s1, r1s_ref[...], r2s_ref[...])
        rvar = jnp.exp(2.0 * rls)
        nll_r = 0.5 * (2.0 * rls + (vr0 - rmu) ** 2 / (rvar + 1e-12))
        v_r = (m0 <= _GATE_RADAR) & is_t

        parts = []
        for b in range(nb):
            bm = bid == b
            parts.append(jnp.asarray(0.0, f32))
            parts.append(jnp.asarray(0.0, f32))
            for s in (1, 2):
                vrb = v_r & bm & (sid == s)
                parts.append(jnp.sum(jnp.where(vrb, nll_r, 0.0)))
                parts.append(jnp.sum(vrb.astype(f32)))
        out_ref[...] = jnp.concatenate(
            [jnp.stack(parts), jnp.stack(reg_parts)]).reshape(1, 1, nout)

        @pl.when(jnp.any(is_lidar & is_t))
        def _lidar():
            ms = [m0]
            gxk = [jnp.sum(jnp.where(one0, gx_sel, 0.0), axis=1, keepdims=True)]
            gyk = [jnp.sum(jnp.where(one0, gy_sel, 0.0), axis=1, keepdims=True)]
            run = jnp.where(one0, inf, running)
            for k in range(1, _ASSOC_TOPK):
                m = jnp.min(run, axis=1, keepdims=True)
                hit = run == m
                idx = jnp.min(jnp.where(hit, iota, nk), axis=1, keepdims=True)
                one = iota == idx
                gxk.append(jnp.sum(jnp.where(one, gx_sel, 0.0), axis=1,
                                   keepdims=True))
                gyk.append(jnp.sum(jnp.where(one, gy_sel, 0.0), axis=1,
                                   keepdims=True))
                ms.append(m)
                if k + 1 < _ASSOC_TOPK:
                    run = jnp.where(one, inf, run)

            z = [-(m * m) / tau_sq for m in ms]
            z0 = z[0]
            es = [jnp.exp(zk - z0) for zk in z]
            denom = ((es[0] + es[1]) + es[2]) + (es[3] + es[4])
            ex = jnp.zeros_like(z0)
            ey = jnp.zeros_like(z0)
            for k in range(_ASSOC_TOPK):
                w = es[k] / denom
                ex = ex + w * gxk[k]
                ey = ey + w * gyk[k]

            var0 = jnp.exp(2.0 * lls[:, 0:1])
            var1 = jnp.exp(2.0 * lls[:, 1:2])
            nll_l = 0.5 * (
                (2.0 * lls[:, 0:1] + (ex - lmu[:, 0:1]) ** 2 / (var0 + 1e-12))
                + (2.0 * lls[:, 1:2] + (ey - lmu[:, 1:2]) ** 2 / (var1 + 1e-12)))
            v_l = (m0 <= _GATE_LIDAR) & is_lidar & is_t

            parts_l = list(parts)
            for b in range(nb):
                bm = bid == b
                vlb = v_l & bm
                parts_l[6 * b] = jnp.sum(jnp.where(vlb, nll_l, 0.0))
                parts_l[6 * b + 1] = jnp.sum(vlb.astype(f32))
            out_ref[...] = jnp.concatenate(
                [jnp.stack(parts_l), jnp.stack(reg_parts)]).reshape(1, 1, nout)


def kernel(lidar_mu, lidar_log_sigma, r1_mu, r1_log_sigma, r2_mu,
           r2_log_sigma, x_t, x_tp1, pose_by_frame, frame_id_t, batch_id_t,
           sensor_id_t, batch_id_tp1, sensor_id_tp1):
    n_t = lidar_mu.shape[0]
    nk = x_tp1.shape[0]
    nb = pose_by_frame.shape[0]
    t_idx = _WINDOW - 1

    blk = 64
    while n_t % blk:
        blk //= 2
    grid = n_t // blk
    nout = 6 * nb + 3

    f32 = jnp.float32
    i32 = jnp.int32
    fid0 = frame_id_t.astype(i32)
    sid0 = sensor_id_t.astype(i32)
    cls = jnp.where(fid0 != _WINDOW - 1, 2, jnp.where(sid0 == 0, 0, 1))
    perm = jnp.argsort(cls, stable=True)
    rxy = x_t[perm, :2].astype(f32)
    kvr = x_tp1[:, _IDX_VR].reshape(1, nk).astype(f32)
    fid = fid0[perm].reshape(n_t, 1)
    bidt = batch_id_t.astype(i32)[perm].reshape(n_t, 1)
    sidt = sid0[perm].reshape(n_t, 1)
    kbid = batch_id_tp1.reshape(1, nk).astype(i32)
    ksid = sensor_id_tp1.reshape(1, nk).astype(i32)
    lidar_mu = lidar_mu[perm]
    lidar_log_sigma = lidar_log_sigma[perm]
    r1_mu = r1_mu[perm]
    r1_log_sigma = r1_log_sigma[perm]
    r2_mu = r2_mu[perm]
    r2_log_sigma = r2_log_sigma[perm]

    gt_rows = []
    for b in range(nb):
        pose_t = pose_by_frame[b, t_idx]
        pose_tp1 = pose_by_frame[b, t_idx + 1]
        g = _se2_apply(_se2_inv(pose_t), _se2_apply(pose_tp1, x_tp1[:, :2]))
        gt_rows.append(g)
    gxs = jnp.stack([g[:, 0] for g in gt_rows]).astype(f32)
    gys = jnp.stack([g[:, 1] for g in gt_rows]).astype(f32)

    row = lambda w: pl.BlockSpec((blk, w), lambda i: (i, 0))
    key = pl.BlockSpec((1, nk), lambda i: (0, 0))
    keyb = pl.BlockSpec((nb, nk), lambda i: (0, 0))

    partials = pl.pallas_call(
        functools.partial(_loss_block_kernel, nb),
        grid=(grid,),
        in_specs=[
            row(2), row(2), row(2), row(1), row(1), row(1), row(1),
            row(1), row(1), row(1),
            keyb, keyb, key, key, key,
        ],
        out_specs=pl.BlockSpec((1, 1, nout), lambda i: (i, 0, 0)),
        out_shape=jax.ShapeDtypeStruct((grid, 1, nout), f32),
    )(lidar_mu.astype(f32), lidar_log_sigma.astype(f32), rxy,
      r1_mu.astype(f32), r1_log_sigma.astype(f32), r2_mu.astype(f32),
      r2_log_sigma.astype(f32), fid, bidt, sidt, gxs, gys, kvr, kbid, ksid)

    p = partials.reshape(grid, nout).sum(axis=0)

    zero = jnp.asarray(0.0, f32)

    def seg(s, c):
        return jnp.where(c > 0, s / jnp.maximum(c, 1.0), 0.0)

    lidar_sum = zero
    lidar_cnt = zero
    r1_sum = zero
    r1_cnt = zero
    r2_sum = zero
    r2_cnt = zero
    for b in range(nb):
        o = 6 * b
        lidar_sum = lidar_sum + seg(p[o + 0], p[o + 1])
        lidar_cnt = lidar_cnt + (p[o + 1] > 0).astype(f32)
        r1_sum = r1_sum + seg(p[o + 2], p[o + 3])
        r1_cnt = r1_cnt + (p[o + 3] > 0).astype(f32)
        r2_sum = r2_sum + seg(p[o + 4], p[o + 5])
        r2_cnt = r2_cnt + (p[o + 5] > 0).astype(f32)
    loss_l = jnp.where(lidar_cnt > 0, lidar_sum / jnp.maximum(lidar_cnt, 1.0), zero)
    loss_r1 = jnp.where(r1_cnt > 0, r1_sum / jnp.maximum(r1_cnt, 1.0), zero)
    loss_r2 = jnp.where(r2_cnt > 0, r2_sum / jnp.maximum(r2_cnt, 1.0), zero)
    o = 6 * nb
    reg = (p[o] / (lidar_log_sigma.size) + p[o + 1] / r1_log_sigma.size
           + p[o + 2] / r2_log_sigma.size)
    total = loss_l + _RADAR_LOSS_WEIGHT * (loss_r1 + loss_r2) + _REG_LAMBDA * reg
    return (total, loss_l, loss_r1, loss_r2, reg)


# speedup vs baseline: 12.4928x; 1.2324x over previous
import functools

import jax
import jax.numpy as jnp
from jax.experimental import pallas as pl

_WINDOW = 4
_IDX_VR = 6
_ASSOC_TOPK = 5
_ASSOC_TAU = 0.5
_GATE_LIDAR = 0.15
_GATE_RADAR = 0.3
_RADAR_LOSS_WEIGHT = 5.0
_REG_LAMBDA = 1e-3


def _se2_apply(p, pts):
    x, y, th = p[0], p[1], p[2]
    c, s = jnp.cos(th), jnp.sin(th)
    R = jnp.stack([jnp.stack([c, -s]), jnp.stack([s, c])])
    return pts @ R.T + jnp.stack([x, y])


def _se2_inv(p):
    x, y, th = p[0], p[1], p[2]
    c, s = jnp.cos(th), jnp.sin(th)
    return jnp.stack([-(c * x + s * y), -(-s * x + c * y), -th])


def _loss_block_kernel(nb, lmu_ref, lls_ref, rxy_ref, r1m_ref, r1s_ref,
                       r2m_ref, r2s_ref, fid_ref, bid_ref, sid_ref,
                       gx_refs_ref, gy_refs_ref, kvr_ref, kbid_ref,
                       ksid_ref, out_ref):
    f32 = jnp.float32
    nk = kvr_ref.shape[1]
    tau_sq = max(_ASSOC_TAU ** 2, 1e-6)
    t_idx = _WINDOW - 1

    lmu = lmu_ref[...]
    lls = lls_ref[...]
    rxy = rxy_ref[...]
    fid = fid_ref[...]
    bid = bid_ref[...]
    sid = sid_ref[...]
    kvr = kvr_ref[...]
    kbid = kbid_ref[...]
    ksid = ksid_ref[...]
    gxs = [gx_refs_ref[b:b + 1, :] for b in range(nb)]
    gys = [gy_refs_ref[b:b + 1, :] for b in range(nb)]

    def r8(x):
        p = x * 65537.0
        return p - (p - x)

    b0 = bid == 0
    b2s = [gxs[b] * gxs[b] + gys[b] * gys[b] for b in range(nb)]
    gxh = [r8(g) for g in gxs]
    gyh = [r8(g) for g in gys]
    if nb == 2:
        gxh_sel = jnp.where(b0, gxh[0], gxh[1])
        gyh_sel = jnp.where(b0, gyh[0], gyh[1])
        b2_sel = jnp.where(b0, b2s[0], b2s[1])

        def gxy_sel():
            return (jnp.where(b0, gxs[0], gxs[1]),
                    jnp.where(b0, gys[0], gys[1]))
    else:
        gxh_sel = jnp.broadcast_to(gxh[0], (lmu.shape[0], nk))
        gyh_sel = jnp.broadcast_to(gyh[0], (lmu.shape[0], nk))
        b2_sel = jnp.broadcast_to(b2s[0], (lmu.shape[0], nk))

        def gxy_sel():
            return (jnp.broadcast_to(gxs[0], (lmu.shape[0], nk)),
                    jnp.broadcast_to(gys[0], (lmu.shape[0], nk)))

    is_lidar = sid == 0
    is_t = fid == t_idx
    nout = 6 * nb + 3

    reg_parts = [jnp.sum(lls * lls), jnp.sum(r1s_ref[...] ** 2),
                 jnp.sum(r2s_ref[...] ** 2)]
    out_ref[...] = jnp.concatenate(
        [jnp.zeros((6 * nb,), f32), jnp.stack(reg_parts)]).reshape(1, 1, nout)

    @pl.when(jnp.any(is_t))
    def _live():
        qx = jnp.where(is_lidar, lmu[:, 0:1], rxy[:, 0:1])
        qy = jnp.where(is_lidar, lmu[:, 1:2], rxy[:, 1:2])

        a2 = qx * qx + qy * qy
        ab = r8(qx) * gxh_sel + r8(qy) * gyh_sel
        sq = jnp.maximum(a2 + b2_sel - 2.0 * ab, 0.0)
        d = jnp.sqrt(sq)

        col_ok = (kbid == bid) & (ksid == sid)
        inf = jnp.float32(jnp.inf)
        running = jnp.where(col_ok, d, inf)

        iota = jax.lax.broadcasted_iota(jnp.int32, (1, nk), 1)
        m0 = jnp.min(running, axis=1, keepdims=True)
        hit0 = running == m0
        idx0 = jnp.min(jnp.where(hit0, iota, nk), axis=1, keepdims=True)
        one0 = iota == idx0
        vr0 = jnp.sum(jnp.where(one0, kvr, 0.0), axis=1, keepdims=True)

        s1 = sid == 1
        rmu = jnp.where(s1, r1m_ref[...], r2m_ref[...])
        rls = jnp.where(s1, r1s_ref[...], r2s_ref[...])
        rvar = jnp.exp(2.0 * rls)
        nll_r = 0.5 * (2.0 * rls + (vr0 - rmu) ** 2 / (rvar + 1e-12))
        v_r = (m0 <= _GATE_RADAR) & is_t

        parts = []
        for b in range(nb):
            bm = bid == b
            parts.append(jnp.asarray(0.0, f32))
            parts.append(jnp.asarray(0.0, f32))
            for s in (1, 2):
                vrb = v_r & bm & (sid == s)
                parts.append(jnp.sum(jnp.where(vrb, nll_r, 0.0)))
                parts.append(jnp.sum(vrb.astype(f32)))
        out_ref[...] = jnp.concatenate(
            [jnp.stack(parts), jnp.stack(reg_parts)]).reshape(1, 1, nout)

        @pl.when(jnp.any(is_lidar & is_t))
        def _lidar():
            gx_sel, gy_sel = gxy_sel()
            ms = [m0]
            gxk = [jnp.sum(jnp.where(one0, gx_sel, 0.0), axis=1, keepdims=True)]
            gyk = [jnp.sum(jnp.where(one0, gy_sel, 0.0), axis=1, keepdims=True)]
            run = jnp.where(one0, inf, running)
            for k in range(1, _ASSOC_TOPK):
                m = jnp.min(run, axis=1, keepdims=True)
                hit = run == m
                idx = jnp.min(jnp.where(hit, iota, nk), axis=1, keepdims=True)
                one = iota == idx
                gxk.append(jnp.sum(jnp.where(one, gx_sel, 0.0), axis=1,
                                   keepdims=True))
                gyk.append(jnp.sum(jnp.where(one, gy_sel, 0.0), axis=1,
                                   keepdims=True))
                ms.append(m)
                if k + 1 < _ASSOC_TOPK:
                    run = jnp.where(one, inf, run)

            z = [-(m * m) / tau_sq for m in ms]
            z0 = z[0]
            es = [jnp.exp(zk - z0) for zk in z]
            denom = ((es[0] + es[1]) + es[2]) + (es[3] + es[4])
            ex = jnp.zeros_like(z0)
            ey = jnp.zeros_like(z0)
            for k in range(_ASSOC_TOPK):
                w = es[k] / denom
                ex = ex + w * gxk[k]
                ey = ey + w * gyk[k]

            var0 = jnp.exp(2.0 * lls[:, 0:1])
            var1 = jnp.exp(2.0 * lls[:, 1:2])
            nll_l = 0.5 * (
                (2.0 * lls[:, 0:1] + (ex - lmu[:, 0:1]) ** 2 / (var0 + 1e-12))
                + (2.0 * lls[:, 1:2] + (ey - lmu[:, 1:2]) ** 2 / (var1 + 1e-12)))
            v_l = (m0 <= _GATE_LIDAR) & is_lidar & is_t

            parts_l = list(parts)
            for b in range(nb):
                bm = bid == b
                vlb = v_l & bm
                parts_l[6 * b] = jnp.sum(jnp.where(vlb, nll_l, 0.0))
                parts_l[6 * b + 1] = jnp.sum(vlb.astype(f32))
            out_ref[...] = jnp.concatenate(
                [jnp.stack(parts_l), jnp.stack(reg_parts)]).reshape(1, 1, nout)


def kernel(lidar_mu, lidar_log_sigma, r1_mu, r1_log_sigma, r2_mu,
           r2_log_sigma, x_t, x_tp1, pose_by_frame, frame_id_t, batch_id_t,
           sensor_id_t, batch_id_tp1, sensor_id_tp1):
    n_t = lidar_mu.shape[0]
    nk = x_tp1.shape[0]
    nb = pose_by_frame.shape[0]
    t_idx = _WINDOW - 1

    blk = 64
    while n_t % blk:
        blk //= 2
    grid = n_t // blk
    nout = 6 * nb + 3

    f32 = jnp.float32
    i32 = jnp.int32
    fid0 = frame_id_t.astype(i32)
    sid0 = sensor_id_t.astype(i32)
    cls = jnp.where(fid0 != _WINDOW - 1, 2, jnp.where(sid0 == 0, 0, 1))
    perm = jnp.argsort(cls, stable=True)
    rxy = x_t[perm, :2].astype(f32)
    kvr = x_tp1[:, _IDX_VR].reshape(1, nk).astype(f32)
    fid = fid0[perm].reshape(n_t, 1)
    bidt = batch_id_t.astype(i32)[perm].reshape(n_t, 1)
    sidt = sid0[perm].reshape(n_t, 1)
    kbid = batch_id_tp1.reshape(1, nk).astype(i32)
    ksid = sensor_id_tp1.reshape(1, nk).astype(i32)
    lidar_mu = lidar_mu[perm]
    lidar_log_sigma = lidar_log_sigma[perm]
    r1_mu = r1_mu[perm]
    r1_log_sigma = r1_log_sigma[perm]
    r2_mu = r2_mu[perm]
    r2_log_sigma = r2_log_sigma[perm]

    gt_rows = []
    for b in range(nb):
        pose_t = pose_by_frame[b, t_idx]
        pose_tp1 = pose_by_frame[b, t_idx + 1]
        g = _se2_apply(_se2_inv(pose_t), _se2_apply(pose_tp1, x_tp1[:, :2]))
        gt_rows.append(g)
    gxs = jnp.stack([g[:, 0] for g in gt_rows]).astype(f32)
    gys = jnp.stack([g[:, 1] for g in gt_rows]).astype(f32)

    row = lambda w: pl.BlockSpec((blk, w), lambda i: (i, 0))
    key = pl.BlockSpec((1, nk), lambda i: (0, 0))
    keyb = pl.BlockSpec((nb, nk), lambda i: (0, 0))

    partials = pl.pallas_call(
        functools.partial(_loss_block_kernel, nb),
        grid=(grid,),
        in_specs=[
            row(2), row(2), row(2), row(1), row(1), row(1), row(1),
            row(1), row(1), row(1),
            keyb, keyb, key, key, key,
        ],
        out_specs=pl.BlockSpec((1, 1, nout), lambda i: (i, 0, 0)),
        out_shape=jax.ShapeDtypeStruct((grid, 1, nout), f32),
    )(lidar_mu.astype(f32), lidar_log_sigma.astype(f32), rxy,
      r1_mu.astype(f32), r1_log_sigma.astype(f32), r2_mu.astype(f32),
      r2_log_sigma.astype(f32), fid, bidt, sidt, gxs, gys, kvr, kbid, ksid)

    p = partials.reshape(grid, nout).sum(axis=0)

    zero = jnp.asarray(0.0, f32)

    def seg(s, c):
        return jnp.where(c > 0, s / jnp.maximum(c, 1.0), 0.0)

    lidar_sum = zero
    lidar_cnt = zero
    r1_sum = zero
    r1_cnt = zero
    r2_sum = zero
    r2_cnt = zero
    for b in range(nb):
        o = 6 * b
        lidar_sum = lidar_sum + seg(p[o + 0], p[o + 1])
        lidar_cnt = lidar_cnt + (p[o + 1] > 0).astype(f32)
        r1_sum = r1_sum + seg(p[o + 2], p[o + 3])
        r1_cnt = r1_cnt + (p[o + 3] > 0).astype(f32)
        r2_sum = r2_sum + seg(p[o + 4], p[o + 5])
        r2_cnt = r2_cnt + (p[o + 5] > 0).astype(f32)
    loss_l = jnp.where(lidar_cnt > 0, lidar_sum / jnp.maximum(lidar_cnt, 1.0), zero)
    loss_r1 = jnp.where(r1_cnt > 0, r1_sum / jnp.maximum(r1_cnt, 1.0), zero)
    loss_r2 = jnp.where(r2_cnt > 0, r2_sum / jnp.maximum(r2_cnt, 1.0), zero)
    o = 6 * nb
    reg = (p[o] / (lidar_log_sigma.size) + p[o + 1] / r1_log_sigma.size
           + p[o + 2] / r2_log_sigma.size)
    total = loss_l + _RADAR_LOSS_WEIGHT * (loss_r1 + loss_r2) + _REG_LAMBDA * reg
    return (total, loss_l, loss_r1, loss_r2, reg)
